# baseline (speedup 1.0000x reference)
import functools

import jax
import jax.numpy as jnp
from jax import lax
from jax.experimental import pallas as pl
from jax.experimental.pallas import tpu as pltpu
from jax.experimental.pallas import tpu_sc as plsc

BATCH = 4
SEQ_LEN = 2048
D_MODEL = 768
_ROWS = BATCH * SEQ_LEN

_NUM_CORES = 2
_NUM_SUBCORES = 16
_NW = _NUM_CORES * _NUM_SUBCORES
_S_PER_W = SEQ_LEN // _NW
_HALF = _S_PER_W // 2
_NHC = BATCH * 2
_NSLOT = 3
_LANES = 16
_D_SLICES = D_MODEL // _LANES


def _body(x_hbm, tok_hbm, pos_hbm, out_hbm, idx_v, tok_v, pos_v,
          gsem, ss0, ss1, psem):
    wid = lax.axis_index("s") * _NUM_CORES + lax.axis_index("c")
    s_base = wid * _S_PER_W

    for b in range(BATCH):
        pltpu.sync_copy(x_hbm.at[b, pl.ds(s_base, _S_PER_W)],
                        idx_v.at[pl.ds(b * _S_PER_W, _S_PER_W)])

    def slot_off(i):
        return lax.rem(i, _NSLOT) * _HALF

    def gather(i):
        return pltpu.make_async_copy(
            tok_hbm.at[idx_v.at[pl.ds(i * _HALF, _HALF)]],
            tok_v.at[pl.ds(slot_off(i), _HALF)], gsem)

    def store(i, sem):
        row_base = lax.div(i, 2) * SEQ_LEN + s_base + lax.rem(i, 2) * _HALF
        return pltpu.make_async_copy(
            tok_v.at[pl.ds(slot_off(i), _HALF)],
            out_hbm.at[pl.ds(row_base, _HALF)], sem)

    def store_parity(i, action):
        par = lax.rem(i, 2)

        @pl.when(par == 0)
        def _():
            action(ss0)

        @pl.when(par == 1)
        def _():
            action(ss1)

    def add_rows(i):
        off = slot_off(i)
        pos_off = lax.rem(i, 2) * _HALF

        def per_row(r, _):
            for j in range(_D_SLICES):
                sl = pl.ds(j * _LANES, _LANES)
                plsc.addupdate(tok_v.at[off + r, sl],
                               pos_v[pos_off + r, sl])
            return 0

        lax.fori_loop(0, _HALF, per_row, 0, unroll=False)

    gather(0).start()
    pos_cp = pltpu.make_async_copy(pos_hbm.at[pl.ds(s_base, _S_PER_W)],
                                   pos_v, psem)
    pos_cp.start()
    pos_cp.wait()

    def step(i, _):
        gather(i).wait()

        @pl.when(i >= 2)
        def _():
            store_parity(i, lambda sem: store(i - 2, sem).wait())

        gather(i + 1).start()
        add_rows(i)
        store_parity(i, lambda sem: store(i, sem).start())
        return 0

    lax.fori_loop(0, _NHC - 1, step, 0, unroll=False)

    last = _NHC - 1
    gather(last).wait()
    add_rows(last)
    store(last, ss1).start()
    store(_NHC - 3, ss1).wait()
    store(_NHC - 2, ss0).wait()
    store(last, ss1).wait()


@functools.partial(
    pl.kernel,
    out_type=jax.ShapeDtypeStruct((_ROWS, D_MODEL), jnp.float32),
    mesh=plsc.VectorSubcoreMesh(core_axis_name="c", subcore_axis_name="s"),
    scratch_types=[
        pltpu.VMEM((_NHC * _HALF,), jnp.int32),
        pltpu.VMEM((_NSLOT * _HALF, D_MODEL), jnp.float32),
        pltpu.VMEM((_S_PER_W, D_MODEL), jnp.float32),
        pltpu.SemaphoreType.DMA,
        pltpu.SemaphoreType.DMA,
        pltpu.SemaphoreType.DMA,
        pltpu.SemaphoreType.DMA,
    ],
)
def _emb_lookup(x_hbm, tok_hbm, pos_hbm, out_hbm, idx_v, tok_v, pos_v,
                gsem, ss0, ss1, psem):
    _body(x_hbm, tok_hbm, pos_hbm, out_hbm, idx_v, tok_v, pos_v,
          gsem, ss0, ss1, psem)


def kernel(x, token_table, position_table):
    x = x.astype(jnp.int32)
    out = _emb_lookup(x, token_table, position_table)
    return out.reshape(BATCH, SEQ_LEN, D_MODEL)


# speedup vs baseline: 1.3672x; 1.3672x over previous
import functools

import jax
import jax.numpy as jnp
from jax import lax
from jax.experimental import pallas as pl
from jax.experimental.pallas import tpu as pltpu
from jax.experimental.pallas import tpu_sc as plsc

BATCH = 4
SEQ_LEN = 2048
D_MODEL = 768
_ROWS = BATCH * SEQ_LEN

_NUM_CORES = 2
_NUM_SUBCORES = 16
_NW = _NUM_CORES * _NUM_SUBCORES
_S_PER_W = SEQ_LEN // _NW
_HALF = _S_PER_W // 2
_SUB = _HALF // 2
_NHC = BATCH * 2
_LANES = 16
_D_SLICES = D_MODEL // _LANES


def _body(x_hbm, tok_hbm, pos_hbm, out_hbm, idx_v, tok_v, pos_v,
          gsem, ssem, psem):
    wid = lax.axis_index("s") * _NUM_CORES + lax.axis_index("c")
    s_base = wid * _S_PER_W

    for b in range(BATCH):
        pltpu.sync_copy(x_hbm.at[b, pl.ds(s_base, _S_PER_W)],
                        idx_v.at[pl.ds(b * _S_PER_W, _S_PER_W)])

    def gather(i):
        off = lax.rem(i, 2) * _HALF
        return pltpu.make_async_copy(
            tok_hbm.at[idx_v.at[pl.ds(i * _HALF, _HALF)]],
            tok_v.at[pl.ds(off, _HALF)], gsem)

    def sub_store(i, half):
        off = lax.rem(i, 2) * _HALF + half * _SUB
        row_base = (lax.div(i, 2) * SEQ_LEN + s_base
                    + lax.rem(i, 2) * _HALF + half * _SUB)
        return pltpu.make_async_copy(
            tok_v.at[pl.ds(off, _SUB)],
            out_hbm.at[pl.ds(row_base, _SUB)], ssem)

    def add_sub(i, half):
        off = lax.rem(i, 2) * _HALF + half * _SUB

        def per_row(r, _):
            for j in range(_D_SLICES):
                sl = pl.ds(j * _LANES, _LANES)
                plsc.addupdate(tok_v.at[off + r, sl], pos_v[off + r, sl])
            return 0

        lax.fori_loop(0, _SUB, per_row, 0, unroll=False)

    def process(i):
        add_sub(i, 0)
        sub_store(i, 0).start()
        add_sub(i, 1)
        sub_store(i, 1).start()
        sub_store(i, 0).wait()
        sub_store(i, 1).wait()

    gather(0).start()
    pos_cp = pltpu.make_async_copy(pos_hbm.at[pl.ds(s_base, _S_PER_W)],
                                   pos_v, psem)
    pos_cp.start()
    pos_cp.wait()

    def step(i, _):
        gather(i).wait()
        gather(i + 1).start()
        process(i)
        return 0

    lax.fori_loop(0, _NHC - 1, step, 0, unroll=False)
    gather(_NHC - 1).wait()
    process(_NHC - 1)


@functools.partial(
    pl.kernel,
    out_type=jax.ShapeDtypeStruct((_ROWS, D_MODEL), jnp.float32),
    mesh=plsc.VectorSubcoreMesh(core_axis_name="c", subcore_axis_name="s"),
    scratch_types=[
        pltpu.VMEM((_NHC * _HALF,), jnp.int32),
        pltpu.VMEM((_S_PER_W, D_MODEL), jnp.float32),
        pltpu.VMEM((_S_PER_W, D_MODEL), jnp.float32),
        pltpu.SemaphoreType.DMA,
        pltpu.SemaphoreType.DMA,
        pltpu.SemaphoreType.DMA,
    ],
)
def _emb_lookup(x_hbm, tok_hbm, pos_hbm, out_hbm, idx_v, tok_v, pos_v,
                gsem, ssem, psem):
    _body(x_hbm, tok_hbm, pos_hbm, out_hbm, idx_v, tok_v, pos_v,
          gsem, ssem, psem)


def kernel(x, token_table, position_table):
    x = x.astype(jnp.int32)
    out = _emb_lookup(x, token_table, position_table)
    return out.reshape(BATCH, SEQ_LEN, D_MODEL)


# speedup vs baseline: 1.3700x; 1.0021x over previous
import functools

import jax
import jax.numpy as jnp
from jax import lax
from jax.experimental import pallas as pl
from jax.experimental.pallas import tpu as pltpu
from jax.experimental.pallas import tpu_sc as plsc

BATCH = 4
SEQ_LEN = 2048
D_MODEL = 768
_ROWS = BATCH * SEQ_LEN

_NUM_CORES = 2
_NUM_SUBCORES = 16
_NW = _NUM_CORES * _NUM_SUBCORES
_S_PER_W = SEQ_LEN // _NW
_HALF = _S_PER_W // 2
_SUB = _HALF // 2
_NHC = BATCH * 2
_LANES = 16
_D_SLICES = D_MODEL // _LANES


def _body(x_hbm, tok_hbm, pos_hbm, out_hbm, idx_v, tok_v, pos_v,
          gsem, ssem, psem):
    wid = lax.axis_index("s") * _NUM_CORES + lax.axis_index("c")
    s_base = wid * _S_PER_W

    for b in range(BATCH):
        pltpu.sync_copy(x_hbm.at[b, pl.ds(s_base, _S_PER_W)],
                        idx_v.at[pl.ds(b * _S_PER_W, _S_PER_W)])

    def gather(i):
        off = lax.rem(i, 2) * _HALF
        return pltpu.make_async_copy(
            tok_hbm.at[idx_v.at[pl.ds(i * _HALF, _HALF)]],
            tok_v.at[pl.ds(off, _HALF)], gsem)

    def sub_store(i, half):
        off = lax.rem(i, 2) * _HALF + half * _SUB
        row_base = (lax.div(i, 2) * SEQ_LEN + s_base
                    + lax.rem(i, 2) * _HALF + half * _SUB)
        return pltpu.make_async_copy(
            tok_v.at[pl.ds(off, _SUB)],
            out_hbm.at[pl.ds(row_base, _SUB)], ssem)

    def add_sub(i, half):
        off = lax.rem(i, 2) * _HALF + half * _SUB

        @plsc.parallel_loop(0, _SUB, step=1, unroll=2)
        def per_row(r):
            for j in range(_D_SLICES):
                sl = pl.ds(j * _LANES, _LANES)
                plsc.addupdate(tok_v.at[off + r, sl], pos_v[off + r, sl])

    def process(i):
        add_sub(i, 0)
        sub_store(i, 0).start()
        add_sub(i, 1)
        sub_store(i, 1).start()
        sub_store(i, 0).wait()
        sub_store(i, 1).wait()

    gather(0).start()
    pos_cp = pltpu.make_async_copy(pos_hbm.at[pl.ds(s_base, _S_PER_W)],
                                   pos_v, psem)
    pos_cp.start()
    pos_cp.wait()

    def step(i, _):
        gather(i).wait()
        gather(i + 1).start()
        process(i)
        return 0

    lax.fori_loop(0, _NHC - 1, step, 0, unroll=False)
    gather(_NHC - 1).wait()
    process(_NHC - 1)


@functools.partial(
    pl.kernel,
    out_type=jax.ShapeDtypeStruct((_ROWS, D_MODEL), jnp.float32),
    mesh=plsc.VectorSubcoreMesh(core_axis_name="c", subcore_axis_name="s"),
    scratch_types=[
        pltpu.VMEM((_NHC * _HALF,), jnp.int32),
        pltpu.VMEM((_S_PER_W, D_MODEL), jnp.float32),
        pltpu.VMEM((_S_PER_W, D_MODEL), jnp.float32),
        pltpu.SemaphoreType.DMA,
        pltpu.SemaphoreType.DMA,
        pltpu.SemaphoreType.DMA,
    ],
)
def _emb_lookup(x_hbm, tok_hbm, pos_hbm, out_hbm, idx_v, tok_v, pos_v,
                gsem, ssem, psem):
    _body(x_hbm, tok_hbm, pos_hbm, out_hbm, idx_v, tok_v, pos_v,
          gsem, ssem, psem)


def kernel(x, token_table, position_table):
    x = x.astype(jnp.int32)
    out = _emb_lookup(x, token_table, position_table)
    return out.reshape(BATCH, SEQ_LEN, D_MODEL)
